# baseline (speedup 1.0000x reference)
import functools

import jax
import jax.numpy as jnp
from jax import lax
from jax.experimental import pallas as pl
from jax.experimental.pallas import tpu as pltpu
from jax.experimental.pallas import tpu_sc as plsc

FT_IN = 22528
FT_OUT = 1024
NE = 8
FC0_OUT = 16
L2 = 15
FC1_OUT = 32
B = 16384
H = FT_OUT // 2

_NW = 32
_CH = 8
_NBUF = 4


def _make_sc_body(nbw):
  nchunk = nbw // _CH

  def _sc_gather_body(idx0_hbm, idx1_hbm, ftw_hbm, psqt_hbm,
                      acc0_hbm, acc1_hbm, pd_hbm,
                      idx0_v, idx1_v, rows0_v, rows1_v, pr0_v, pr1_v,
                      pk0_v, pk1_v, pd_v,
                      gsem0, gsem1, gsem2, gsem3, wsem0, wsem1):
    gsem = [gsem0, gsem1, gsem2, gsem3]
    wsem = [wsem0, wsem1]
    u32 = jnp.uint32
    himask = jnp.uint32(0xFFFF0000)
    wid = lax.axis_index("s") * 2 + lax.axis_index("c")
    base = wid * nbw
    pltpu.sync_copy(idx0_hbm.at[pl.ds(base, nbw)], idx0_v)
    pltpu.sync_copy(idx1_hbm.at[pl.ds(base, nbw)], idx1_v)

    def fire_g(ch, q):
        pltpu.async_copy(ftw_hbm.at[idx0_v.at[pl.ds(ch * _CH, _CH)]],
                         rows0_v.at[q], gsem[q])
        pltpu.async_copy(ftw_hbm.at[idx1_v.at[pl.ds(ch * _CH, _CH)]],
                         rows1_v.at[q], gsem[q])
        pltpu.async_copy(psqt_hbm.at[idx0_v.at[pl.ds(ch * _CH, _CH)]],
                         pr0_v.at[q], gsem[q])
        pltpu.async_copy(psqt_hbm.at[idx1_v.at[pl.ds(ch * _CH, _CH)]],
                         pr1_v.at[q], gsem[q])

    def drain_g(q):
        dums = ftw_hbm.at[pl.ds(0, _CH)]
        dump = psqt_hbm.at[pl.ds(0, _CH)]
        pltpu.make_async_copy(dums, rows0_v.at[q], gsem[q]).wait()
        pltpu.make_async_copy(dums, rows1_v.at[q], gsem[q]).wait()
        pltpu.make_async_copy(dump, pr0_v.at[q], gsem[q]).wait()
        pltpu.make_async_copy(dump, pr1_v.at[q], gsem[q]).wait()

    def fire_w(ch, p):
        off = base + ch * _CH
        pltpu.async_copy(pk0_v.at[p], acc0_hbm.at[pl.ds(off, _CH)], wsem[p])
        pltpu.async_copy(pk1_v.at[p], acc1_hbm.at[pl.ds(off, _CH)], wsem[p])
        pltpu.async_copy(pd_v.at[p], pd_hbm.at[pl.ds(off, _CH)], wsem[p])

    def drain_w(p):
        dum5 = acc0_hbm.at[pl.ds(0, _CH)]
        dump = pd_hbm.at[pl.ds(0, _CH)]
        pltpu.make_async_copy(dum5, pk0_v.at[p], wsem[p]).wait()
        pltpu.make_async_copy(dum5, pk1_v.at[p], wsem[p]).wait()
        pltpu.make_async_copy(dump, pd_v.at[p], wsem[p]).wait()

    def compute(q, p):
        def tok(t, carry):
            for m in range(FT_OUT // 32):
                for rv, pk in ((rows0_v, pk0_v), (rows1_v, pk1_v)):
                    x = lax.bitcast_convert_type(
                        rv.at[q][t, pl.ds(32 * m, 16)][...], u32)
                    y = lax.bitcast_convert_type(
                        rv.at[q][t, pl.ds(32 * m + 16, 16)][...], u32)
                    w = (x >> 16) | (y & himask)
                    pk.at[p][t, pl.ds(16 * m, 16)] = (
                        lax.bitcast_convert_type(w, jnp.int32))
            for k in range(8):
                d = (pr0_v.at[q][t, pl.ds(16 * k, 16)][...]
                     - pr1_v.at[q][t, pl.ds(16 * k, 16)][...]) * 0.5
                pd_v.at[p][t, pl.ds(16 * k, 16)] = d
            return carry

        lax.fori_loop(0, _CH, tok, 0)

    fire_g(0, 0)
    fire_g(1, 1)

    def body(j, carry):
        for q in range(4):
            ch = j * 4 + q
            p = q % 2
            drain_g(q)
            if q < 2:
                @pl.when(j > 0)
                def _():
                    drain_w(p)
            else:
                drain_w(p)
            compute(q, p)
            fire_w(ch, p)
            q2 = (q + 2) % 4
            if q < 2:
                fire_g(ch + 2, q2)
            else:
                @pl.when(j < (nchunk // 4 - 1))
                def _():
                    fire_g(ch + 2, q2)
        return carry

    lax.fori_loop(0, nchunk // 4, body, 0)
    drain_w(0)
    drain_w(1)

  return _sc_gather_body


def _sc_gather(idx0, idx1, ftw, psqt_pad):
    n = idx0.shape[0]
    nbw = n // _NW
    mesh = plsc.VectorSubcoreMesh(core_axis_name="c", subcore_axis_name="s")
    f32 = jnp.float32
    i32 = jnp.int32
    run = functools.partial(
        pl.kernel,
        mesh=mesh,
        out_type=[
            jax.ShapeDtypeStruct((n, FT_OUT // 2), i32),
            jax.ShapeDtypeStruct((n, FT_OUT // 2), i32),
            jax.ShapeDtypeStruct((n, 128), f32),
        ],
        scratch_types=[
            pltpu.VMEM((nbw,), i32),
            pltpu.VMEM((nbw,), i32),
            pltpu.VMEM((_NBUF, _CH, FT_OUT), f32),
            pltpu.VMEM((_NBUF, _CH, FT_OUT), f32),
            pltpu.VMEM((_NBUF, _CH, 128), f32),
            pltpu.VMEM((_NBUF, _CH, 128), f32),
            pltpu.VMEM((2, _CH, FT_OUT // 2), i32),
            pltpu.VMEM((2, _CH, FT_OUT // 2), i32),
            pltpu.VMEM((2, _CH, 128), f32),
            pltpu.SemaphoreType.DMA,
            pltpu.SemaphoreType.DMA,
            pltpu.SemaphoreType.DMA,
            pltpu.SemaphoreType.DMA,
            pltpu.SemaphoreType.DMA,
            pltpu.SemaphoreType.DMA,
        ],
    )(_make_sc_body(nbw))
    return run(idx0, idx1, ftw, psqt_pad)


_BLK = 512


def _tc_mlp_body(acc0_ref, acc1_ref, pd_ref, bucket_ref, ba_ref, bb_ref,
                 w0a0_ref, w0b0_ref, w0a1_ref, w0b1_ref,
                 w1sqr_ref, w1rel_ref, w2_ref, sk_ref,
                 b0_ref, b1_ref, b2_ref, out_ref):
    f32 = jnp.float32
    himask = jnp.int32(-65536)

    def halves(u):
        lo = lax.bitcast_convert_type(u << 16, f32)
        hi = lax.bitcast_convert_type(u & himask, f32)
        return lo, hi

    def pair(a, bias):
        s0 = jnp.clip(a[:, :256] + bias[:, :256], 0.0, 127.0)
        s1 = jnp.clip(a[:, 256:] + bias[:, 256:], 0.0, 127.0)
        return s0 * s1

    lo0, hi0 = halves(acc0_ref[...])
    lo1, hi1 = halves(acc1_ref[...])
    ba = ba_ref[...]
    bb = bb_ref[...]
    o0 = (jax.lax.dot(pair(lo0, ba), w0a0_ref[...], preferred_element_type=f32)
          + jax.lax.dot(pair(hi0, bb), w0b0_ref[...], preferred_element_type=f32)
          + jax.lax.dot(pair(lo1, ba), w0a1_ref[...], preferred_element_type=f32)
          + jax.lax.dot(pair(hi1, bb), w0b1_ref[...], preferred_element_type=f32)
          + b0_ref[...])
    sqr = jnp.clip(o0 * o0 * (1.0 / float(1 << 19)), 0.0, 127.0)
    rel = jnp.clip(o0 * (1.0 / float(1 << 6)), 0.0, 127.0)
    o1 = (jax.lax.dot(sqr, w1sqr_ref[...], preferred_element_type=f32)
          + jax.lax.dot(rel, w1rel_ref[...], preferred_element_type=f32)
          + b1_ref[...])
    ac1 = jnp.clip(o1 * (1.0 / float(1 << 6)), 0.0, 127.0)
    sc_all = (jax.lax.dot(ac1, w2_ref[...], preferred_element_type=f32)
              + b2_ref[...])
    skip_all = jax.lax.dot(o0, sk_ref[...], preferred_element_type=f32)
    tot = sc_all + skip_all

    bucket = bucket_ref[...]
    lanes8 = lax.broadcasted_iota(jnp.int32, (tot.shape[0], 8), 1)
    onehot8 = (lanes8 == bucket).astype(f32)
    positional = jnp.sum(tot * onehot8, axis=1, keepdims=True)

    lanes128 = lax.broadcasted_iota(jnp.int32, (tot.shape[0], 128), 1)
    onehot128 = (lanes128 == bucket).astype(f32)
    psqt_val = jnp.sum(pd_ref[...] * onehot128, axis=1, keepdims=True)

    out_ref[...] = (psqt_val + positional) * (1.0 / 16.0)


def _tc_mlp(acc0p, acc1p, pd, bucket2d, ba, bb,
            w0a0, w0b0, w0a1, w0b1, w1sqr, w1rel, w2t, skm, b0f, b1f, b2f):
    f32 = jnp.float32
    n = acc0p.shape[0]
    grid = (n // _BLK,)
    full = lambda shape: pl.BlockSpec(shape, lambda i: (0, 0))
    blk = lambda w: pl.BlockSpec((_BLK, w), lambda i: (i, 0))
    return pl.pallas_call(
        _tc_mlp_body,
        grid=grid,
        in_specs=[
            blk(H), blk(H), blk(128), blk(1),
            full((1, H)), full((1, H)),
            full((256, 128)), full((256, 128)),
            full((256, 128)), full((256, 128)),
            full((128, 256)), full((128, 256)),
            full((256, 8)), full((128, 8)),
            full((1, 128)), full((1, 256)), full((1, 8)),
        ],
        out_specs=blk(1),
        out_shape=jax.ShapeDtypeStruct((n, 1), f32),
    )(acc0p, acc1p, pd, bucket2d, ba, bb,
      w0a0, w0b0, w0a1, w0b1, w1sqr, w1rel, w2t, skm, b0f, b1f, b2f)


_NSPLIT = 4


def kernel(w_feats, w_offsets, b_feats, b_offsets, stm, bucket,
           ft_w, ft_bias, psqt_w, fc0_w, fc0_b, fc1_w, fc1_b, fc2_w, fc2_b):
    f32 = jnp.float32
    wf = w_feats.astype(jnp.int32)
    bf = b_feats.astype(jnp.int32)
    stm_i = stm.astype(jnp.int32)
    bucket_i = bucket.astype(jnp.int32)

    swap = stm_i == 1
    idx0 = jnp.where(swap, bf, wf)
    idx1 = jnp.where(swap, wf, bf)

    psqt_pad = jnp.pad(psqt_w, ((0, 0), (0, 128 - NE)))

    eye = jnp.eye(NE, dtype=f32)
    w0 = fc0_w.reshape(NE * FC0_OUT, FT_OUT).T * (1.0 / 128.0)
    c = jnp.arange(H)
    la = 32 * (c // 16) + c % 16
    ba = ft_bias[la].reshape(1, H)
    bb = ft_bias[la + 16].reshape(1, H)
    la256 = la[:256]
    w0a0 = w0[la256]
    w0b0 = w0[la256 + 16]
    w0a1 = w0[la256 + 512]
    w0b1 = w0[la256 + 528]
    sqr_w = jnp.pad(jnp.transpose(fc1_w[:, :, :L2], (0, 2, 1)),
                    ((0, 0), (0, FC0_OUT - L2), (0, 0)))
    rel_w = jnp.pad(jnp.transpose(fc1_w[:, :, L2:2 * L2], (0, 2, 1)),
                    ((0, 0), (0, FC0_OUT - L2), (0, 0)))
    w1sqr = jnp.einsum('ejo,ef->ejfo', sqr_w, eye).reshape(128, 256)
    w1rel = jnp.einsum('ejo,ef->ejfo', rel_w, eye).reshape(128, 256)
    w2t = jnp.einsum('ei,ef->eif', fc2_w[:, 0, :], eye).reshape(256, 8)
    spot = jnp.zeros((FC0_OUT,), f32).at[L2].set(9600.0 / 8128.0)
    skm = jnp.einsum('j,ef->ejf', spot, eye).reshape(128, 8)
    b0f = fc0_b.reshape(1, 128)
    b1f = fc1_b.reshape(1, 256)
    b2f = fc2_b.reshape(1, 8)

    ns = _NSPLIT
    sb = B // ns
    bucket2d = bucket_i.reshape(B, 1)
    gathered = [
        _sc_gather(idx0[h * sb:(h + 1) * sb], idx1[h * sb:(h + 1) * sb],
                   ft_w, psqt_pad)
        for h in range(ns)
    ]
    outs = [
        _tc_mlp(acc0p, acc1p, pd, bucket2d[h * sb:(h + 1) * sb], ba, bb,
                w0a0, w0b0, w0a1, w0b1, w1sqr, w1rel, w2t, skm, b0f, b1f, b2f)
        for h, (acc0p, acc1p, pd) in enumerate(gathered)
    ]
    out = jnp.concatenate(outs, axis=0)
    return out[:, 0]


# speedup vs baseline: 1.0789x; 1.0789x over previous
import functools

import jax
import jax.numpy as jnp
from jax import lax
from jax.experimental import pallas as pl
from jax.experimental.pallas import tpu as pltpu
from jax.experimental.pallas import tpu_sc as plsc

FT_IN = 22528
FT_OUT = 1024
NE = 8
FC0_OUT = 16
L2 = 15
FC1_OUT = 32
B = 16384
H = FT_OUT // 2

_NW = 32
_CH = 8
_NBUF = 4


def _make_sc_body(nbw):
  nchunk = nbw // _CH

  def _sc_gather_body(idx0_hbm, idx1_hbm, ftw_hbm, psqt_hbm,
                      acc0_hbm, acc1_hbm, pd_hbm,
                      idx0_v, idx1_v, rows0_v, rows1_v, pr0_v, pr1_v,
                      pk0_v, pk1_v, pd_v,
                      gsem0, gsem1, gsem2, gsem3, wsem0, wsem1):
    gsem = [gsem0, gsem1, gsem2, gsem3]
    wsem = [wsem0, wsem1]
    u32 = jnp.uint32
    himask = jnp.uint32(0xFFFF0000)
    wid = lax.axis_index("s") * 2 + lax.axis_index("c")
    base = wid * nbw
    pltpu.sync_copy(idx0_hbm.at[pl.ds(base, nbw)], idx0_v)
    pltpu.sync_copy(idx1_hbm.at[pl.ds(base, nbw)], idx1_v)

    def fire_g(ch, q):
        pltpu.async_copy(ftw_hbm.at[idx0_v.at[pl.ds(ch * _CH, _CH)]],
                         rows0_v.at[q], gsem[q])
        pltpu.async_copy(ftw_hbm.at[idx1_v.at[pl.ds(ch * _CH, _CH)]],
                         rows1_v.at[q], gsem[q])
        pltpu.async_copy(psqt_hbm.at[idx0_v.at[pl.ds(ch * _CH, _CH)]],
                         pr0_v.at[q], gsem[q])
        pltpu.async_copy(psqt_hbm.at[idx1_v.at[pl.ds(ch * _CH, _CH)]],
                         pr1_v.at[q], gsem[q])

    def drain_g(q):
        dums = ftw_hbm.at[pl.ds(0, _CH)]
        dump = psqt_hbm.at[pl.ds(0, _CH)]
        pltpu.make_async_copy(dums, rows0_v.at[q], gsem[q]).wait()
        pltpu.make_async_copy(dums, rows1_v.at[q], gsem[q]).wait()
        pltpu.make_async_copy(dump, pr0_v.at[q], gsem[q]).wait()
        pltpu.make_async_copy(dump, pr1_v.at[q], gsem[q]).wait()

    def fire_w(ch, p):
        off = base + ch * _CH
        pltpu.async_copy(pk0_v.at[p], acc0_hbm.at[pl.ds(off, _CH)], wsem[p])
        pltpu.async_copy(pk1_v.at[p], acc1_hbm.at[pl.ds(off, _CH)], wsem[p])
        pltpu.async_copy(pd_v.at[p], pd_hbm.at[pl.ds(off, _CH)], wsem[p])

    def drain_w(p):
        dum5 = acc0_hbm.at[pl.ds(0, _CH)]
        dump = pd_hbm.at[pl.ds(0, _CH)]
        pltpu.make_async_copy(dum5, pk0_v.at[p], wsem[p]).wait()
        pltpu.make_async_copy(dum5, pk1_v.at[p], wsem[p]).wait()
        pltpu.make_async_copy(dump, pd_v.at[p], wsem[p]).wait()

    def compute(q, p):
        def tok(t, carry):
            for m in range(FT_OUT // 32):
                for rv, pk in ((rows0_v, pk0_v), (rows1_v, pk1_v)):
                    x = lax.bitcast_convert_type(
                        rv.at[q][t, pl.ds(32 * m, 16)][...], u32)
                    y = lax.bitcast_convert_type(
                        rv.at[q][t, pl.ds(32 * m + 16, 16)][...], u32)
                    w = (x >> 16) | (y & himask)
                    pk.at[p][t, pl.ds(16 * m, 16)] = (
                        lax.bitcast_convert_type(w, jnp.int32))
            for k in range(8):
                d = (pr0_v.at[q][t, pl.ds(16 * k, 16)][...]
                     - pr1_v.at[q][t, pl.ds(16 * k, 16)][...]) * 0.5
                pd_v.at[p][t, pl.ds(16 * k, 16)] = d
            return carry

        lax.fori_loop(0, _CH, tok, 0)

    fire_g(0, 0)
    fire_g(1, 1)

    def body(j, carry):
        for q in range(4):
            ch = j * 4 + q
            p = q % 2
            drain_g(q)
            if q < 2:
                @pl.when(j > 0)
                def _():
                    drain_w(p)
            else:
                drain_w(p)
            compute(q, p)
            fire_w(ch, p)
            q2 = (q + 2) % 4
            if q < 2:
                fire_g(ch + 2, q2)
            else:
                @pl.when(j < (nchunk // 4 - 1))
                def _():
                    fire_g(ch + 2, q2)
        return carry

    lax.fori_loop(0, nchunk // 4, body, 0)
    drain_w(0)
    drain_w(1)

  return _sc_gather_body


def _sc_gather(idx0, idx1, ftw, psqt_pad):
    n = idx0.shape[0]
    nbw = n // _NW
    mesh = plsc.VectorSubcoreMesh(core_axis_name="c", subcore_axis_name="s")
    f32 = jnp.float32
    i32 = jnp.int32
    run = functools.partial(
        pl.kernel,
        mesh=mesh,
        out_type=[
            jax.ShapeDtypeStruct((n, FT_OUT // 2), i32),
            jax.ShapeDtypeStruct((n, FT_OUT // 2), i32),
            jax.ShapeDtypeStruct((n, 128), f32),
        ],
        scratch_types=[
            pltpu.VMEM((nbw,), i32),
            pltpu.VMEM((nbw,), i32),
            pltpu.VMEM((_NBUF, _CH, FT_OUT), f32),
            pltpu.VMEM((_NBUF, _CH, FT_OUT), f32),
            pltpu.VMEM((_NBUF, _CH, 128), f32),
            pltpu.VMEM((_NBUF, _CH, 128), f32),
            pltpu.VMEM((2, _CH, FT_OUT // 2), i32),
            pltpu.VMEM((2, _CH, FT_OUT // 2), i32),
            pltpu.VMEM((2, _CH, 128), f32),
            pltpu.SemaphoreType.DMA,
            pltpu.SemaphoreType.DMA,
            pltpu.SemaphoreType.DMA,
            pltpu.SemaphoreType.DMA,
            pltpu.SemaphoreType.DMA,
            pltpu.SemaphoreType.DMA,
        ],
    )(_make_sc_body(nbw))
    return run(idx0, idx1, ftw, psqt_pad)


_BLK = 1024


def _tc_mlp_body(acc0_ref, acc1_ref, pd_ref, bucket_ref, ba_ref, bb_ref,
                 w0a0_ref, w0b0_ref, w0a1_ref, w0b1_ref,
                 w1sqr_ref, w1rel_ref, w2_ref, sk_ref,
                 b0_ref, b1_ref, b2_ref, out_ref):
    f32 = jnp.float32
    himask = jnp.int32(-65536)

    def halves(u):
        lo = lax.bitcast_convert_type(u << 16, f32)
        hi = lax.bitcast_convert_type(u & himask, f32)
        return lo, hi

    def pair(a, bias):
        s0 = jnp.clip(a[:, :256] + bias[:, :256], 0.0, 127.0)
        s1 = jnp.clip(a[:, 256:] + bias[:, 256:], 0.0, 127.0)
        return s0 * s1

    lo0, hi0 = halves(acc0_ref[...])
    lo1, hi1 = halves(acc1_ref[...])
    ba = ba_ref[...]
    bb = bb_ref[...]
    o0 = (jax.lax.dot(pair(lo0, ba), w0a0_ref[...], preferred_element_type=f32)
          + jax.lax.dot(pair(hi0, bb), w0b0_ref[...], preferred_element_type=f32)
          + jax.lax.dot(pair(lo1, ba), w0a1_ref[...], preferred_element_type=f32)
          + jax.lax.dot(pair(hi1, bb), w0b1_ref[...], preferred_element_type=f32)
          + b0_ref[...])
    sqr = jnp.clip(o0 * o0 * (1.0 / float(1 << 19)), 0.0, 127.0)
    rel = jnp.clip(o0 * (1.0 / float(1 << 6)), 0.0, 127.0)
    o1 = (jax.lax.dot(sqr, w1sqr_ref[...], preferred_element_type=f32)
          + jax.lax.dot(rel, w1rel_ref[...], preferred_element_type=f32)
          + b1_ref[...])
    ac1 = jnp.clip(o1 * (1.0 / float(1 << 6)), 0.0, 127.0)
    sc_all = (jax.lax.dot(ac1, w2_ref[...], preferred_element_type=f32)
              + b2_ref[...])
    skip_all = jax.lax.dot(o0, sk_ref[...], preferred_element_type=f32)
    tot = sc_all + skip_all

    bucket = bucket_ref[...]
    lanes8 = lax.broadcasted_iota(jnp.int32, (tot.shape[0], 8), 1)
    onehot8 = (lanes8 == bucket).astype(f32)
    positional = jnp.sum(tot * onehot8, axis=1, keepdims=True)

    lanes128 = lax.broadcasted_iota(jnp.int32, (tot.shape[0], 128), 1)
    onehot128 = (lanes128 == bucket).astype(f32)
    psqt_val = jnp.sum(pd_ref[...] * onehot128, axis=1, keepdims=True)

    out_ref[...] = (psqt_val + positional) * (1.0 / 16.0)


def _tc_mlp(acc0p, acc1p, pd, bucket2d, ba, bb,
            w0a0, w0b0, w0a1, w0b1, w1sqr, w1rel, w2t, skm, b0f, b1f, b2f):
    f32 = jnp.float32
    n = acc0p.shape[0]
    grid = (n // _BLK,)
    full = lambda shape: pl.BlockSpec(shape, lambda i: (0, 0))
    blk = lambda w: pl.BlockSpec((_BLK, w), lambda i: (i, 0))
    return pl.pallas_call(
        _tc_mlp_body,
        grid=grid,
        in_specs=[
            blk(H), blk(H), blk(128), blk(1),
            full((1, H)), full((1, H)),
            full((256, 128)), full((256, 128)),
            full((256, 128)), full((256, 128)),
            full((128, 256)), full((128, 256)),
            full((256, 8)), full((128, 8)),
            full((1, 128)), full((1, 256)), full((1, 8)),
        ],
        out_specs=blk(1),
        out_shape=jax.ShapeDtypeStruct((n, 1), f32),
    )(acc0p, acc1p, pd, bucket2d, ba, bb,
      w0a0, w0b0, w0a1, w0b1, w1sqr, w1rel, w2t, skm, b0f, b1f, b2f)


_NSPLIT = 2


def kernel(w_feats, w_offsets, b_feats, b_offsets, stm, bucket,
           ft_w, ft_bias, psqt_w, fc0_w, fc0_b, fc1_w, fc1_b, fc2_w, fc2_b):
    f32 = jnp.float32
    wf = w_feats.astype(jnp.int32)
    bf = b_feats.astype(jnp.int32)
    stm_i = stm.astype(jnp.int32)
    bucket_i = bucket.astype(jnp.int32)

    swap = stm_i == 1
    idx0 = jnp.where(swap, bf, wf)
    idx1 = jnp.where(swap, wf, bf)

    psqt_pad = jnp.pad(psqt_w, ((0, 0), (0, 128 - NE)))

    eye = jnp.eye(NE, dtype=f32)
    w0 = fc0_w.reshape(NE * FC0_OUT, FT_OUT).T * (1.0 / 128.0)
    c = jnp.arange(H)
    la = 32 * (c // 16) + c % 16
    ba = ft_bias[la].reshape(1, H)
    bb = ft_bias[la + 16].reshape(1, H)
    la256 = la[:256]
    w0a0 = w0[la256]
    w0b0 = w0[la256 + 16]
    w0a1 = w0[la256 + 512]
    w0b1 = w0[la256 + 528]
    sqr_w = jnp.pad(jnp.transpose(fc1_w[:, :, :L2], (0, 2, 1)),
                    ((0, 0), (0, FC0_OUT - L2), (0, 0)))
    rel_w = jnp.pad(jnp.transpose(fc1_w[:, :, L2:2 * L2], (0, 2, 1)),
                    ((0, 0), (0, FC0_OUT - L2), (0, 0)))
    w1sqr = jnp.einsum('ejo,ef->ejfo', sqr_w, eye).reshape(128, 256)
    w1rel = jnp.einsum('ejo,ef->ejfo', rel_w, eye).reshape(128, 256)
    w2t = jnp.einsum('ei,ef->eif', fc2_w[:, 0, :], eye).reshape(256, 8)
    spot = jnp.zeros((FC0_OUT,), f32).at[L2].set(9600.0 / 8128.0)
    skm = jnp.einsum('j,ef->ejf', spot, eye).reshape(128, 8)
    b0f = fc0_b.reshape(1, 128)
    b1f = fc1_b.reshape(1, 256)
    b2f = fc2_b.reshape(1, 8)

    ns = _NSPLIT
    sb = B // ns
    bucket2d = bucket_i.reshape(B, 1)
    gathered = [
        _sc_gather(idx0[h * sb:(h + 1) * sb], idx1[h * sb:(h + 1) * sb],
                   ft_w, psqt_pad)
        for h in range(ns)
    ]
    outs = [
        _tc_mlp(acc0p, acc1p, pd, bucket2d[h * sb:(h + 1) * sb], ba, bb,
                w0a0, w0b0, w0a1, w0b1, w1sqr, w1rel, w2t, skm, b0f, b1f, b2f)
        for h, (acc0p, acc1p, pd) in enumerate(gathered)
    ]
    out = jnp.concatenate(outs, axis=0)
    return out[:, 0]


# speedup vs baseline: 1.0826x; 1.0034x over previous
import functools

import jax
import jax.numpy as jnp
from jax import lax
from jax.experimental import pallas as pl
from jax.experimental.pallas import tpu as pltpu
from jax.experimental.pallas import tpu_sc as plsc

FT_IN = 22528
FT_OUT = 1024
NE = 8
FC0_OUT = 16
L2 = 15
FC1_OUT = 32
B = 16384
H = FT_OUT // 2

_NW = 32
_CH = 8
_NBUF = 4


def _make_sc_body(nbw):
  nchunk = nbw // _CH

  def _sc_gather_body(idx0_hbm, idx1_hbm, ftw_hbm, psqt_hbm,
                      acc0_hbm, acc1_hbm, pd_hbm,
                      idx0_v, idx1_v, rows0_v, rows1_v, pr0_v, pr1_v,
                      pk0_v, pk1_v, pd_v,
                      gsem0, gsem1, gsem2, gsem3, wsem0, wsem1):
    gsem = [gsem0, gsem1, gsem2, gsem3]
    wsem = [wsem0, wsem1]
    u32 = jnp.uint32
    himask = jnp.uint32(0xFFFF0000)
    wid = lax.axis_index("s") * 2 + lax.axis_index("c")
    base = wid * nbw
    pltpu.sync_copy(idx0_hbm.at[pl.ds(base, nbw)], idx0_v)
    pltpu.sync_copy(idx1_hbm.at[pl.ds(base, nbw)], idx1_v)

    def fire_g(ch, q):
        pltpu.async_copy(ftw_hbm.at[idx0_v.at[pl.ds(ch * _CH, _CH)]],
                         rows0_v.at[q], gsem[q])
        pltpu.async_copy(ftw_hbm.at[idx1_v.at[pl.ds(ch * _CH, _CH)]],
                         rows1_v.at[q], gsem[q])
        pltpu.async_copy(psqt_hbm.at[idx0_v.at[pl.ds(ch * _CH, _CH)]],
                         pr0_v.at[q], gsem[q])
        pltpu.async_copy(psqt_hbm.at[idx1_v.at[pl.ds(ch * _CH, _CH)]],
                         pr1_v.at[q], gsem[q])

    def drain_g(q):
        dums = ftw_hbm.at[pl.ds(0, _CH)]
        dump = psqt_hbm.at[pl.ds(0, _CH)]
        pltpu.make_async_copy(dums, rows0_v.at[q], gsem[q]).wait()
        pltpu.make_async_copy(dums, rows1_v.at[q], gsem[q]).wait()
        pltpu.make_async_copy(dump, pr0_v.at[q], gsem[q]).wait()
        pltpu.make_async_copy(dump, pr1_v.at[q], gsem[q]).wait()

    def fire_w(ch, p):
        off = base + ch * _CH
        pltpu.async_copy(pk0_v.at[p], acc0_hbm.at[pl.ds(off, _CH)], wsem[p])
        pltpu.async_copy(pk1_v.at[p], acc1_hbm.at[pl.ds(off, _CH)], wsem[p])
        pltpu.async_copy(pd_v.at[p], pd_hbm.at[pl.ds(off, _CH)], wsem[p])

    def drain_w(p):
        dum5 = acc0_hbm.at[pl.ds(0, _CH)]
        dump = pd_hbm.at[pl.ds(0, _CH)]
        pltpu.make_async_copy(dum5, pk0_v.at[p], wsem[p]).wait()
        pltpu.make_async_copy(dum5, pk1_v.at[p], wsem[p]).wait()
        pltpu.make_async_copy(dump, pd_v.at[p], wsem[p]).wait()

    def compute(q, p):
        def tok(t, carry):
            for m in range(FT_OUT // 32):
                for rv, pk in ((rows0_v, pk0_v), (rows1_v, pk1_v)):
                    x = lax.bitcast_convert_type(
                        rv.at[q][t, pl.ds(32 * m, 16)][...], u32)
                    y = lax.bitcast_convert_type(
                        rv.at[q][t, pl.ds(32 * m + 16, 16)][...], u32)
                    w = (x >> 16) | (y & himask)
                    pk.at[p][t, pl.ds(16 * m, 16)] = (
                        lax.bitcast_convert_type(w, jnp.int32))
            for k in range(8):
                d = (pr0_v.at[q][t, pl.ds(16 * k, 16)][...]
                     - pr1_v.at[q][t, pl.ds(16 * k, 16)][...]) * 0.5
                pd_v.at[p][t, pl.ds(16 * k, 16)] = d
            return carry

        lax.fori_loop(0, _CH, tok, 0)

    fire_g(0, 0)
    fire_g(1, 1)

    def body(j, carry):
        for q in range(4):
            ch = j * 4 + q
            p = q % 2
            drain_g(q)
            if q < 2:
                @pl.when(j > 0)
                def _():
                    drain_w(p)
            else:
                drain_w(p)
            compute(q, p)
            fire_w(ch, p)
            q2 = (q + 2) % 4
            if q < 2:
                fire_g(ch + 2, q2)
            else:
                @pl.when(j < (nchunk // 4 - 1))
                def _():
                    fire_g(ch + 2, q2)
        return carry

    lax.fori_loop(0, nchunk // 4, body, 0)
    drain_w(0)
    drain_w(1)

  return _sc_gather_body


def _sc_gather(idx0, idx1, ftw, psqt_pad):
    n = idx0.shape[0]
    nbw = n // _NW
    mesh = plsc.VectorSubcoreMesh(core_axis_name="c", subcore_axis_name="s")
    f32 = jnp.float32
    i32 = jnp.int32
    run = functools.partial(
        pl.kernel,
        mesh=mesh,
        out_type=[
            jax.ShapeDtypeStruct((n, FT_OUT // 2), i32),
            jax.ShapeDtypeStruct((n, FT_OUT // 2), i32),
            jax.ShapeDtypeStruct((n, 128), f32),
        ],
        scratch_types=[
            pltpu.VMEM((nbw,), i32),
            pltpu.VMEM((nbw,), i32),
            pltpu.VMEM((_NBUF, _CH, FT_OUT), f32),
            pltpu.VMEM((_NBUF, _CH, FT_OUT), f32),
            pltpu.VMEM((_NBUF, _CH, 128), f32),
            pltpu.VMEM((_NBUF, _CH, 128), f32),
            pltpu.VMEM((2, _CH, FT_OUT // 2), i32),
            pltpu.VMEM((2, _CH, FT_OUT // 2), i32),
            pltpu.VMEM((2, _CH, 128), f32),
            pltpu.SemaphoreType.DMA,
            pltpu.SemaphoreType.DMA,
            pltpu.SemaphoreType.DMA,
            pltpu.SemaphoreType.DMA,
            pltpu.SemaphoreType.DMA,
            pltpu.SemaphoreType.DMA,
        ],
    )(_make_sc_body(nbw))
    return run(idx0, idx1, ftw, psqt_pad)


_BLK = 1024


def _tc_mlp_body(acc0_ref, acc1_ref, pd_ref, bucket_ref, ba_ref, bb_ref,
                 w0a0_ref, w0b0_ref, w0a1_ref, w0b1_ref,
                 w1sqr_ref, w1rel_ref, w2_ref, sk_ref,
                 b0_ref, b1_ref, b2_ref, out_ref):
    f32 = jnp.float32
    himask = jnp.int32(-65536)

    def halves(u):
        lo = lax.bitcast_convert_type(u << 16, f32)
        hi = lax.bitcast_convert_type(u & himask, f32)
        return lo, hi

    def pair(a, bias):
        s0 = jnp.clip(a[:, :256] + bias[:, :256], 0.0, 127.0)
        s1 = jnp.clip(a[:, 256:] + bias[:, 256:], 0.0, 127.0)
        return s0 * s1

    lo0, hi0 = halves(acc0_ref[...])
    lo1, hi1 = halves(acc1_ref[...])
    ba = ba_ref[...]
    bb = bb_ref[...]
    bf = jnp.bfloat16
    o0 = (jax.lax.dot(pair(lo0, ba).astype(bf), w0a0_ref[...],
                      preferred_element_type=f32)
          + jax.lax.dot(pair(hi0, bb).astype(bf), w0b0_ref[...],
                        preferred_element_type=f32)
          + jax.lax.dot(pair(lo1, ba).astype(bf), w0a1_ref[...],
                        preferred_element_type=f32)
          + jax.lax.dot(pair(hi1, bb).astype(bf), w0b1_ref[...],
                        preferred_element_type=f32)
          + b0_ref[...])
    sqr = jnp.clip(o0 * o0 * (1.0 / float(1 << 19)), 0.0, 127.0)
    rel = jnp.clip(o0 * (1.0 / float(1 << 6)), 0.0, 127.0)
    o1 = (jax.lax.dot(sqr, w1sqr_ref[...], preferred_element_type=f32)
          + jax.lax.dot(rel, w1rel_ref[...], preferred_element_type=f32)
          + b1_ref[...])
    ac1 = jnp.clip(o1 * (1.0 / float(1 << 6)), 0.0, 127.0)
    sc_all = (jax.lax.dot(ac1, w2_ref[...], preferred_element_type=f32)
              + b2_ref[...])
    skip_all = jax.lax.dot(o0, sk_ref[...], preferred_element_type=f32)
    tot = sc_all + skip_all

    bucket = bucket_ref[...]
    lanes8 = lax.broadcasted_iota(jnp.int32, (tot.shape[0], 8), 1)
    onehot8 = (lanes8 == bucket).astype(f32)
    positional = jnp.sum(tot * onehot8, axis=1, keepdims=True)

    lanes128 = lax.broadcasted_iota(jnp.int32, (tot.shape[0], 128), 1)
    onehot128 = (lanes128 == bucket).astype(f32)
    psqt_val = jnp.sum(pd_ref[...] * onehot128, axis=1, keepdims=True)

    out_ref[...] = (psqt_val + positional) * (1.0 / 16.0)


def _tc_mlp(acc0p, acc1p, pd, bucket2d, ba, bb,
            w0a0, w0b0, w0a1, w0b1, w1sqr, w1rel, w2t, skm, b0f, b1f, b2f):
    f32 = jnp.float32
    n = acc0p.shape[0]
    grid = (n // _BLK,)
    full = lambda shape: pl.BlockSpec(shape, lambda i: (0, 0))
    blk = lambda w: pl.BlockSpec((_BLK, w), lambda i: (i, 0))
    return pl.pallas_call(
        _tc_mlp_body,
        grid=grid,
        in_specs=[
            blk(H), blk(H), blk(128), blk(1),
            full((1, H)), full((1, H)),
            full((256, 128)), full((256, 128)),
            full((256, 128)), full((256, 128)),
            full((128, 256)), full((128, 256)),
            full((256, 8)), full((128, 8)),
            full((1, 128)), full((1, 256)), full((1, 8)),
        ],
        out_specs=blk(1),
        out_shape=jax.ShapeDtypeStruct((n, 1), f32),
    )(acc0p, acc1p, pd, bucket2d, ba, bb,
      w0a0, w0b0, w0a1, w0b1, w1sqr, w1rel, w2t, skm, b0f, b1f, b2f)


_NSPLIT = 2


def kernel(w_feats, w_offsets, b_feats, b_offsets, stm, bucket,
           ft_w, ft_bias, psqt_w, fc0_w, fc0_b, fc1_w, fc1_b, fc2_w, fc2_b):
    f32 = jnp.float32
    wf = w_feats.astype(jnp.int32)
    bf = b_feats.astype(jnp.int32)
    stm_i = stm.astype(jnp.int32)
    bucket_i = bucket.astype(jnp.int32)

    swap = stm_i == 1
    idx0 = jnp.where(swap, bf, wf)
    idx1 = jnp.where(swap, wf, bf)

    psqt_pad = jnp.pad(psqt_w, ((0, 0), (0, 128 - NE)))

    eye = jnp.eye(NE, dtype=f32)
    w0 = fc0_w.reshape(NE * FC0_OUT, FT_OUT).T * (1.0 / 128.0)
    c = jnp.arange(H)
    la = 32 * (c // 16) + c % 16
    ba = ft_bias[la].reshape(1, H)
    bb = ft_bias[la + 16].reshape(1, H)
    la256 = la[:256]
    w0a0 = w0[la256].astype(jnp.bfloat16)
    w0b0 = w0[la256 + 16].astype(jnp.bfloat16)
    w0a1 = w0[la256 + 512].astype(jnp.bfloat16)
    w0b1 = w0[la256 + 528].astype(jnp.bfloat16)
    sqr_w = jnp.pad(jnp.transpose(fc1_w[:, :, :L2], (0, 2, 1)),
                    ((0, 0), (0, FC0_OUT - L2), (0, 0)))
    rel_w = jnp.pad(jnp.transpose(fc1_w[:, :, L2:2 * L2], (0, 2, 1)),
                    ((0, 0), (0, FC0_OUT - L2), (0, 0)))
    w1sqr = jnp.einsum('ejo,ef->ejfo', sqr_w, eye).reshape(128, 256)
    w1rel = jnp.einsum('ejo,ef->ejfo', rel_w, eye).reshape(128, 256)
    w2t = jnp.einsum('ei,ef->eif', fc2_w[:, 0, :], eye).reshape(256, 8)
    spot = jnp.zeros((FC0_OUT,), f32).at[L2].set(9600.0 / 8128.0)
    skm = jnp.einsum('j,ef->ejf', spot, eye).reshape(128, 8)
    b0f = fc0_b.reshape(1, 128)
    b1f = fc1_b.reshape(1, 256)
    b2f = fc2_b.reshape(1, 8)

    ns = _NSPLIT
    sb = B // ns
    bucket2d = bucket_i.reshape(B, 1)
    gathered = [
        _sc_gather(idx0[h * sb:(h + 1) * sb], idx1[h * sb:(h + 1) * sb],
                   ft_w, psqt_pad)
        for h in range(ns)
    ]
    outs = [
        _tc_mlp(acc0p, acc1p, pd, bucket2d[h * sb:(h + 1) * sb], ba, bb,
                w0a0, w0b0, w0a1, w0b1, w1sqr, w1rel, w2t, skm, b0f, b1f, b2f)
        for h, (acc0p, acc1p, pd) in enumerate(gathered)
    ]
    out = jnp.concatenate(outs, axis=0)
    return out[:, 0]
